# baseline (speedup 1.0000x reference)
import functools

import jax
import jax.numpy as jnp
from jax import lax
from jax.experimental import pallas as pl
from jax.experimental.pallas import tpu as pltpu
from jax.experimental.pallas import tpu_sc as plsc

_B = 16384
_L = 50
_NH = 64
_NC = 2
_NS = 16
_NW = _NC * _NS
_RPW = _B // _NW
_G = 8
_NCHUNK = _RPW // _G
_CLEN = _G * _L
_NVEC = _CLEN // 16
_WLEN = _RPW * _L

_mesh = plsc.VectorSubcoreMesh(core_axis_name="c", subcore_axis_name="s")


@functools.partial(
    pl.kernel,
    out_type=jax.ShapeDtypeStruct((_B, _NH), jnp.bfloat16),
    mesh=_mesh,
    scratch_types=[
        pltpu.VMEM((_WLEN,), jnp.int32),
        pltpu.VMEM((_CLEN, _NH), jnp.bfloat16),
        pltpu.VMEM((_CLEN, _NH), jnp.bfloat16),
        pltpu.VMEM((_G, _NH), jnp.bfloat16),
        pltpu.SemaphoreType.DMA,
        pltpu.SemaphoreType.DMA,
    ],
    compiler_params=pltpu.CompilerParams(use_tc_tiling_on_sc=False),
)
def _pool(ctx_hbm, table_hbm, out_hbm, idx_v, rows0, rows1, out_v, sem0, sem1):
    wid = lax.axis_index("s") * _NC + lax.axis_index("c")
    base = wid * _RPW
    pltpu.sync_copy(ctx_hbm.at[pl.ds(base * _L, _WLEN)], idx_v)
    bufs = (rows0, rows1)
    sems = (sem0, sem1)

    def fire(g, buf, sem):
        off = g * _CLEN
        for k in range(_NVEC):
            vec = idx_v[pl.ds(off + k * 16, 16)]
            pltpu.async_copy(
                table_hbm.at[vec], buf.at[pl.ds(k * 16, 16), :], sem
            )

    def drain(buf, sem):
        pltpu.make_async_copy(
            table_hbm.at[pl.ds(0, _CLEN), :], buf, sem
        ).wait()

    def compute(g, buf):
        def row_body(i, carry):
            r0 = i * _L
            accs = [jnp.full((32,), -jnp.inf, jnp.bfloat16) for _ in range(2)]
            for j in range(_L):
                for c2 in range(2):
                    vals = buf[r0 + j, pl.ds(c2 * 32, 32)]
                    accs[c2] = jnp.maximum(accs[c2], vals)
            for c2 in range(2):
                out_v[i, pl.ds(c2 * 32, 32)] = accs[c2]
            return carry

        lax.fori_loop(0, _G, row_body, 0)
        pltpu.sync_copy(out_v, out_hbm.at[pl.ds(base + g * _G, _G), :])

    fire(0, bufs[0], sems[0])

    def body(g2, carry):
        for b2 in range(2):
            g = g2 * 2 + b2
            drain(bufs[b2], sems[b2])

            @pl.when(g + 1 < _NCHUNK)
            def _next():
                fire(g + 1, bufs[1 - b2], sems[1 - b2])

            compute(g, bufs[b2])
        return carry

    lax.fori_loop(0, _NCHUNK // 2, body, 0)


def _fc_body(v_ref, w_ref, b_ref, o_ref):
    vf = v_ref[:, :].astype(jnp.float32)
    o_ref[:, :] = (
        lax.dot_general(
            vf, w_ref[:, :],
            dimension_numbers=(((1,), (1,)), ((), ())),
            preferred_element_type=jnp.float32,
        )
        + b_ref[:, :]
    )


_FC_BLK = 2048


def _fc(v, W, b):
    return pl.pallas_call(
        _fc_body,
        grid=(_B // _FC_BLK,),
        in_specs=[
            pl.BlockSpec((_FC_BLK, _NH), lambda i: (i, 0)),
            pl.BlockSpec((_NH, _NH), lambda i: (0, 0)),
            pl.BlockSpec((1, _NH), lambda i: (0, 0)),
        ],
        out_specs=pl.BlockSpec((_FC_BLK, _NH), lambda i: (i, 0)),
        out_shape=jax.ShapeDtypeStruct((_B, _NH), jnp.float32),
    )(v, W, b.reshape(1, _NH))


def kernel(context, table, W, b):
    tbf = table.astype(jnp.bfloat16).at[0].set(jnp.bfloat16(0.0))
    v = _pool(context.reshape(_B * _L), tbf)
    return _fc(v, W, b)


# speedup vs baseline: 1.5932x; 1.5932x over previous
import functools

import jax
import jax.numpy as jnp
from jax import lax
from jax.experimental import pallas as pl
from jax.experimental.pallas import tpu as pltpu
from jax.experimental.pallas import tpu_sc as plsc

_B = 16384
_L = 50
_NH = 64
_NC = 2
_NS = 16
_NW = _NC * _NS
_RPW = _B // _NW
_G = 8
_NCHUNK = _RPW // _G
_CLEN = _G * _L
_NVEC = _CLEN // 16
_WLEN = _RPW * _L

_mesh = plsc.VectorSubcoreMesh(core_axis_name="c", subcore_axis_name="s")

_DNUMS = lax.GatherDimensionNumbers(
    offset_dims=(), collapsed_slice_dims=(0,), start_index_map=(0,)
)


@functools.partial(
    pl.kernel,
    out_type=jax.ShapeDtypeStruct((_B, _NH), jnp.float32),
    mesh=_mesh,
    scratch_types=[
        pltpu.VMEM((_WLEN,), jnp.int32),
        pltpu.VMEM((_CLEN, _NH), jnp.float32),
        pltpu.VMEM((_CLEN, _NH), jnp.float32),
        pltpu.VMEM((_G, _NH), jnp.float32),
        pltpu.SemaphoreType.DMA,
        pltpu.SemaphoreType.DMA,
    ],
    compiler_params=pltpu.CompilerParams(use_tc_tiling_on_sc=False),
)
def _pool(ctx_hbm, table_hbm, out_hbm, idx_v, rows0, rows1, out_v, sem0, sem1):
    wid = lax.axis_index("s") * _NC + lax.axis_index("c")
    base = wid * _RPW
    pltpu.sync_copy(ctx_hbm.at[pl.ds(base * _L, _WLEN)], idx_v)
    bufs = (rows0, rows1)
    sems = (sem0, sem1)

    def fire(g, buf, sem):
        off = g * _CLEN
        for k in range(_NVEC):
            vec = idx_v[pl.ds(off + k * 16, 16)]
            pltpu.async_copy(
                table_hbm.at[vec], buf.at[pl.ds(k * 16, 16), :], sem
            )

    def drain(buf, sem):
        pltpu.make_async_copy(
            table_hbm.at[pl.ds(0, _CLEN), :], buf, sem
        ).wait()

    def compute(g, buf):
        off = g * _CLEN
        pad = jnp.zeros((16,), jnp.int32)
        for k in range(_NVEC):
            eq = idx_v[pl.ds(off + k * 16, 16)] == 0
            pad = pad | jnp.where(eq, jnp.int32(1), jnp.int32(0))
        anypad = pad[0]
        for t in range(1, 16):
            anypad = anypad | pad[t]

        @pl.when(anypad == 0)
        def _fast():
            def row_body(i, carry):
                r0 = i * _L
                accs = [jnp.full((16,), -jnp.inf, jnp.float32)
                        for _ in range(4)]
                for j in range(_L):
                    for c4 in range(4):
                        vals = buf[r0 + j, pl.ds(c4 * 16, 16)]
                        accs[c4] = jnp.maximum(accs[c4], vals)
                for c4 in range(4):
                    out_v[i, pl.ds(c4 * 16, 16)] = accs[c4]
                return carry

            lax.fori_loop(0, _G, row_body, 0)

        @pl.when(anypad != 0)
        def _slow():
            def row_body(i, carry):
                r0 = i * _L
                accs = [jnp.full((16,), -jnp.inf, jnp.float32)
                        for _ in range(4)]
                for j in range(_L):
                    r = off + r0 + j
                    base16 = (r // 16) * 16
                    lane = jnp.full((16, 1), r - base16, jnp.int32)
                    ivec = idx_v[pl.ds(base16, 16)]
                    splat = lax.gather(
                        ivec, lane, _DNUMS, (1,),
                        mode=lax.GatherScatterMode.PROMISE_IN_BOUNDS,
                    )
                    scale = jnp.where(
                        splat == 0, jnp.float32(0.0), jnp.float32(1.0)
                    )
                    for c4 in range(4):
                        vals = buf[r0 + j, pl.ds(c4 * 16, 16)]
                        accs[c4] = jnp.maximum(accs[c4], vals * scale)
                for c4 in range(4):
                    out_v[i, pl.ds(c4 * 16, 16)] = accs[c4]
                return carry

            lax.fori_loop(0, _G, row_body, 0)

        pltpu.sync_copy(out_v, out_hbm.at[pl.ds(base + g * _G, _G), :])

    fire(0, bufs[0], sems[0])

    def body(g2, carry):
        for b2 in range(2):
            g = g2 * 2 + b2
            drain(bufs[b2], sems[b2])

            @pl.when(g + 1 < _NCHUNK)
            def _next():
                fire(g + 1, bufs[1 - b2], sems[1 - b2])

            compute(g, bufs[b2])
        return carry

    lax.fori_loop(0, _NCHUNK // 2, body, 0)


def _fc_body(v_ref, w_ref, b_ref, o_ref):
    o_ref[:, :] = (
        lax.dot_general(
            v_ref[:, :], w_ref[:, :],
            dimension_numbers=(((1,), (1,)), ((), ())),
            preferred_element_type=jnp.float32,
        )
        + b_ref[:, :]
    )


_FC_BLK = 2048


def _fc(v, W, b):
    return pl.pallas_call(
        _fc_body,
        grid=(_B // _FC_BLK,),
        in_specs=[
            pl.BlockSpec((_FC_BLK, _NH), lambda i: (i, 0)),
            pl.BlockSpec((_NH, _NH), lambda i: (0, 0)),
            pl.BlockSpec((1, _NH), lambda i: (0, 0)),
        ],
        out_specs=pl.BlockSpec((_FC_BLK, _NH), lambda i: (i, 0)),
        out_shape=jax.ShapeDtypeStruct((_B, _NH), jnp.float32),
    )(v, W, b.reshape(1, _NH))


def kernel(context, table, W, b):
    v = _pool(context.reshape(_B * _L), table)
    return _fc(v, W, b)


# speedup vs baseline: 1.6016x; 1.0053x over previous
import functools

import jax
import jax.numpy as jnp
from jax import lax
from jax.experimental import pallas as pl
from jax.experimental.pallas import tpu as pltpu
from jax.experimental.pallas import tpu_sc as plsc

_B = 16384
_L = 50
_NH = 64
_NC = 2
_NS = 16
_NW = _NC * _NS
_RPW = _B // _NW
_G = 8
_NCHUNK = _RPW // _G
_CLEN = _G * _L
_NVEC = _CLEN // 16
_WLEN = _RPW * _L

_mesh = plsc.VectorSubcoreMesh(core_axis_name="c", subcore_axis_name="s")

_DNUMS = lax.GatherDimensionNumbers(
    offset_dims=(), collapsed_slice_dims=(0,), start_index_map=(0,)
)


@functools.partial(
    pl.kernel,
    out_type=jax.ShapeDtypeStruct((_B, _NH), jnp.float32),
    mesh=_mesh,
    scratch_types=[
        pltpu.VMEM((_WLEN,), jnp.int32),
        pltpu.VMEM((_CLEN, _NH), jnp.float32),
        pltpu.VMEM((_CLEN, _NH), jnp.float32),
        pltpu.VMEM((_G, _NH), jnp.float32),
        pltpu.VMEM((_G, _NH), jnp.float32),
        pltpu.SemaphoreType.DMA,
        pltpu.SemaphoreType.DMA,
        pltpu.SemaphoreType.DMA,
        pltpu.SemaphoreType.DMA,
    ],
    compiler_params=pltpu.CompilerParams(use_tc_tiling_on_sc=False),
)
def _pool(ctx_hbm, table_hbm, out_hbm, idx_v, rows0, rows1, out0, out1,
          sem0, sem1, semo0, semo1):
    wid = lax.axis_index("s") * _NC + lax.axis_index("c")
    base = wid * _RPW
    pltpu.sync_copy(ctx_hbm.at[pl.ds(base * _L, _WLEN)], idx_v)
    bufs = (rows0, rows1)
    sems = (sem0, sem1)
    obufs = (out0, out1)
    osems = (semo0, semo1)

    def fire(g, buf, sem):
        off = g * _CLEN
        for k in range(_NVEC):
            vec = idx_v[pl.ds(off + k * 16, 16)]
            pltpu.async_copy(
                table_hbm.at[vec], buf.at[pl.ds(k * 16, 16), :], sem
            )

    def drain(buf, sem):
        pltpu.make_async_copy(
            table_hbm.at[pl.ds(0, _CLEN), :], buf, sem
        ).wait()

    def compute(g, buf, out_v, semo):
        @pl.when(g >= 2)
        def _reuse():
            pltpu.make_async_copy(
                out_v, out_hbm.at[pl.ds(0, _G), :], semo
            ).wait()

        off = g * _CLEN
        pad = jnp.zeros((16,), jnp.int32)
        for k in range(_NVEC):
            eq = idx_v[pl.ds(off + k * 16, 16)] == 0
            pad = pad | jnp.where(eq, jnp.int32(1), jnp.int32(0))
        anypad = pad[0]
        for t in range(1, 16):
            anypad = anypad | pad[t]

        @pl.when(anypad == 0)
        def _fast():
            def row_body(i, carry):
                r0 = i * _L
                accs = [jnp.full((16,), -jnp.inf, jnp.float32)
                        for _ in range(4)]
                for j in range(_L):
                    for c4 in range(4):
                        vals = buf[r0 + j, pl.ds(c4 * 16, 16)]
                        accs[c4] = jnp.maximum(accs[c4], vals)
                for c4 in range(4):
                    out_v[i, pl.ds(c4 * 16, 16)] = accs[c4]
                return carry

            lax.fori_loop(0, _G, row_body, 0)

        @pl.when(anypad != 0)
        def _slow():
            def row_body(i, carry):
                r0 = i * _L
                accs = [jnp.full((16,), -jnp.inf, jnp.float32)
                        for _ in range(4)]
                for j in range(_L):
                    r = off + r0 + j
                    base16 = (r // 16) * 16
                    lane = jnp.full((16, 1), r - base16, jnp.int32)
                    ivec = idx_v[pl.ds(base16, 16)]
                    splat = lax.gather(
                        ivec, lane, _DNUMS, (1,),
                        mode=lax.GatherScatterMode.PROMISE_IN_BOUNDS,
                    )
                    scale = jnp.where(
                        splat == 0, jnp.float32(0.0), jnp.float32(1.0)
                    )
                    for c4 in range(4):
                        vals = buf[r0 + j, pl.ds(c4 * 16, 16)]
                        accs[c4] = jnp.maximum(accs[c4], vals * scale)
                for c4 in range(4):
                    out_v[i, pl.ds(c4 * 16, 16)] = accs[c4]
                return carry

            lax.fori_loop(0, _G, row_body, 0)

        pltpu.async_copy(out_v, out_hbm.at[pl.ds(base + g * _G, _G), :], semo)

    fire(0, bufs[0], sems[0])

    def body(g2, carry):
        for b2 in range(2):
            g = g2 * 2 + b2
            drain(bufs[b2], sems[b2])

            @pl.when(g + 1 < _NCHUNK)
            def _next():
                fire(g + 1, bufs[1 - b2], sems[1 - b2])

            compute(g, bufs[b2], obufs[b2], osems[b2])
        return carry

    lax.fori_loop(0, _NCHUNK // 2, body, 0)
    for p in range(2):
        pltpu.make_async_copy(
            obufs[p], out_hbm.at[pl.ds(0, _G), :], osems[p]
        ).wait()


def _fc_body(v_ref, w_ref, b_ref, o_ref):
    o_ref[:, :] = (
        lax.dot_general(
            v_ref[:, :], w_ref[:, :],
            dimension_numbers=(((1,), (1,)), ((), ())),
            preferred_element_type=jnp.float32,
        )
        + b_ref[:, :]
    )


_FC_BLK = 2048


def _fc(v, W, b):
    return pl.pallas_call(
        _fc_body,
        grid=(_B // _FC_BLK,),
        in_specs=[
            pl.BlockSpec((_FC_BLK, _NH), lambda i: (i, 0)),
            pl.BlockSpec((_NH, _NH), lambda i: (0, 0)),
            pl.BlockSpec((1, _NH), lambda i: (0, 0)),
        ],
        out_specs=pl.BlockSpec((_FC_BLK, _NH), lambda i: (i, 0)),
        out_shape=jax.ShapeDtypeStruct((_B, _NH), jnp.float32),
    )(v, W, b.reshape(1, _NH))


def kernel(context, table, W, b):
    v = _pool(context.reshape(_B * _L), table)
    return _fc(v, W, b)


# speedup vs baseline: 1.6171x; 1.0096x over previous
import functools

import jax
import jax.numpy as jnp
from jax import lax
from jax.experimental import pallas as pl
from jax.experimental.pallas import tpu as pltpu
from jax.experimental.pallas import tpu_sc as plsc

_B = 16384
_L = 50
_NH = 64
_NC = 2
_NS = 16
_NW = _NC * _NS
_RPW = _B // _NW
_G = 8
_NCHUNK = _RPW // _G
_CLEN = _G * _L
_NVEC = _CLEN // 16
_SLEN = 80
_NSTREAM = _CLEN // _SLEN
_WLEN = _RPW * _L

_mesh = plsc.VectorSubcoreMesh(core_axis_name="c", subcore_axis_name="s")

_DNUMS = lax.GatherDimensionNumbers(
    offset_dims=(), collapsed_slice_dims=(0,), start_index_map=(0,)
)


@functools.partial(
    pl.kernel,
    out_type=jax.ShapeDtypeStruct((_B, _NH), jnp.float32),
    mesh=_mesh,
    scratch_types=[
        pltpu.VMEM((_WLEN,), jnp.int32),
        pltpu.VMEM((_CLEN, _NH), jnp.float32),
        pltpu.VMEM((_CLEN, _NH), jnp.float32),
        pltpu.VMEM((_G, _NH), jnp.float32),
        pltpu.VMEM((_G, _NH), jnp.float32),
        pltpu.SemaphoreType.DMA,
        pltpu.SemaphoreType.DMA,
        pltpu.SemaphoreType.DMA,
        pltpu.SemaphoreType.DMA,
    ],
    compiler_params=pltpu.CompilerParams(use_tc_tiling_on_sc=False),
)
def _pool(ctx_hbm, table_hbm, out_hbm, idx_v, rows0, rows1, out0, out1,
          sem0, sem1, semo0, semo1):
    wid = lax.axis_index("s") * _NC + lax.axis_index("c")
    base = wid * _RPW
    pltpu.sync_copy(ctx_hbm.at[pl.ds(base * _L, _WLEN)], idx_v)
    bufs = (rows0, rows1)
    sems = (sem0, sem1)
    obufs = (out0, out1)
    osems = (semo0, semo1)

    def fire(g, buf, sem):
        off = g * _CLEN
        for k in range(_NSTREAM):
            pltpu.async_copy(
                table_hbm.at[idx_v.at[pl.ds(off + k * _SLEN, _SLEN)]],
                buf.at[pl.ds(k * _SLEN, _SLEN), :],
                sem,
            )

    def drain(buf, sem):
        pltpu.make_async_copy(
            table_hbm.at[pl.ds(0, _CLEN), :], buf, sem
        ).wait()

    def compute(g, buf, out_v, semo):
        @pl.when(g >= 2)
        def _reuse():
            pltpu.make_async_copy(
                out_v, out_hbm.at[pl.ds(0, _G), :], semo
            ).wait()

        off = g * _CLEN
        pad = jnp.zeros((16,), jnp.int32)
        for k in range(_NVEC):
            eq = idx_v[pl.ds(off + k * 16, 16)] == 0
            pad = pad | jnp.where(eq, jnp.int32(1), jnp.int32(0))
        anypad = pad[0]
        for t in range(1, 16):
            anypad = anypad | pad[t]

        @pl.when(anypad == 0)
        def _fast():
            def row_body(i, carry):
                r0 = i * _L
                accs = [jnp.full((16,), -jnp.inf, jnp.float32)
                        for _ in range(4)]
                for j in range(_L):
                    for c4 in range(4):
                        vals = buf[r0 + j, pl.ds(c4 * 16, 16)]
                        accs[c4] = jnp.maximum(accs[c4], vals)
                for c4 in range(4):
                    out_v[i, pl.ds(c4 * 16, 16)] = accs[c4]
                return carry

            lax.fori_loop(0, _G, row_body, 0)

        @pl.when(anypad != 0)
        def _slow():
            def row_body(i, carry):
                r0 = i * _L
                accs = [jnp.full((16,), -jnp.inf, jnp.float32)
                        for _ in range(4)]
                for j in range(_L):
                    r = off + r0 + j
                    base16 = (r // 16) * 16
                    lane = jnp.full((16, 1), r - base16, jnp.int32)
                    ivec = idx_v[pl.ds(base16, 16)]
                    splat = lax.gather(
                        ivec, lane, _DNUMS, (1,),
                        mode=lax.GatherScatterMode.PROMISE_IN_BOUNDS,
                    )
                    scale = jnp.where(
                        splat == 0, jnp.float32(0.0), jnp.float32(1.0)
                    )
                    for c4 in range(4):
                        vals = buf[r0 + j, pl.ds(c4 * 16, 16)]
                        accs[c4] = jnp.maximum(accs[c4], vals * scale)
                for c4 in range(4):
                    out_v[i, pl.ds(c4 * 16, 16)] = accs[c4]
                return carry

            lax.fori_loop(0, _G, row_body, 0)

        pltpu.async_copy(out_v, out_hbm.at[pl.ds(base + g * _G, _G), :], semo)

    fire(0, bufs[0], sems[0])

    def body(g2, carry):
        for b2 in range(2):
            g = g2 * 2 + b2
            drain(bufs[b2], sems[b2])

            @pl.when(g + 1 < _NCHUNK)
            def _next():
                fire(g + 1, bufs[1 - b2], sems[1 - b2])

            compute(g, bufs[b2], obufs[b2], osems[b2])
        return carry

    lax.fori_loop(0, _NCHUNK // 2, body, 0)
    for p in range(2):
        pltpu.make_async_copy(
            obufs[p], out_hbm.at[pl.ds(0, _G), :], osems[p]
        ).wait()


def _fc_body(v_ref, w_ref, b_ref, o_ref):
    o_ref[:, :] = (
        lax.dot_general(
            v_ref[:, :], w_ref[:, :],
            dimension_numbers=(((1,), (1,)), ((), ())),
            preferred_element_type=jnp.float32,
        )
        + b_ref[:, :]
    )


_FC_BLK = 2048


def _fc(v, W, b):
    return pl.pallas_call(
        _fc_body,
        grid=(_B // _FC_BLK,),
        in_specs=[
            pl.BlockSpec((_FC_BLK, _NH), lambda i: (i, 0)),
            pl.BlockSpec((_NH, _NH), lambda i: (0, 0)),
            pl.BlockSpec((1, _NH), lambda i: (0, 0)),
        ],
        out_specs=pl.BlockSpec((_FC_BLK, _NH), lambda i: (i, 0)),
        out_shape=jax.ShapeDtypeStruct((_B, _NH), jnp.float32),
    )(v, W, b.reshape(1, _NH))


def kernel(context, table, W, b):
    v = _pool(context.reshape(_B * _L), table)
    return _fc(v, W, b)
